# Initial kernel scaffold; baseline (speedup 1.0000x reference)
import jax
import jax.numpy as jnp
from jax.experimental import pallas as pl


def kernel(items, parents, operations, item_edge_index, op_edge_index, Ws1, bs1, Ws2, bs2, Wp1, bp1, Wp2, bp2, Wch1, bch1, Wch2, bch2, Wo1, bo1, Wo2, bo2, Wc1, bc1, Wc2, bc2, Wc3, bc3):
    raise NotImplementedError("write your pallas kernel here")


# speedup vs baseline: 4.4420x; 4.4420x over previous
import jax
import jax.numpy as jnp
from jax import lax
from jax.experimental import pallas as pl
from jax.experimental.pallas import tpu as pltpu
from jax.experimental.pallas import tpu_sc as plsc

N = 10000
D = 128
E = 320000
NC = 2
NS = 16
CHUNK = 128
EPT = E // NS
NCHUNK = 158
EPT_PAD = NCHUNK * CHUNK
RPT = 624
TAIL_BASE = NS * RPT
TAIL = N - TAIL_BASE
PAR_CHUNKS = 3
PAR_PER_W = PAR_CHUNKS * CHUNK
NPAR_PAD = NC * NS * PAR_PER_W


def _sc_body(items, ops, srcs, dsts, par, zeros, agg, prow,
             isb, idb, rows, pidx, acc, sem_i, sem_g, sem_s):
    c = lax.axis_index("c")
    s = lax.axis_index("s")
    wid = c * NS + s
    base_r = wid * NCHUNK

    pltpu.sync_copy(zeros, acc.at[pl.ds(s * RPT, RPT)])

    @pl.when(s == NS - 1)
    def _():
        pltpu.sync_copy(zeros.at[pl.ds(0, TAIL)], acc.at[pl.ds(TAIL_BASE, TAIL)])

    plsc.subcore_barrier()

    def issue_idx(j, b):
        pltpu.async_copy(srcs.at[base_r + j], isb.at[b], sem_i)
        pltpu.async_copy(dsts.at[base_r + j], idb.at[b], sem_i)

    def wait_idx():
        pltpu.make_async_copy(srcs.at[0], isb.at[0], sem_i).wait()
        pltpu.make_async_copy(dsts.at[0], idb.at[0], sem_i).wait()

    def wait_gather():
        pltpu.make_async_copy(items.at[idb.at[0, 0]], rows.at[0], sem_g).wait()

    def wait_scatter():
        pltpu.make_async_copy(rows.at[0], acc.at[isb.at[0, 0]], sem_s).wait()

    issue_idx(0, 0)

    def chunk(j, carry):
        b = j % 2
        wait_idx()

        @pl.when(c == 0)
        def _():
            pltpu.async_copy(items.at[idb.at[b, 0]], rows.at[b], sem_g)

        @pl.when(c == 1)
        def _():
            pltpu.async_copy(ops.at[idb.at[b, 0]], rows.at[b], sem_g)

        @pl.when(j >= 1)
        def _():
            wait_scatter()

        @pl.when(j + 1 < NCHUNK)
        def _():
            issue_idx(j + 1, 1 - b)

        wait_gather()
        pltpu.async_copy(rows.at[b], acc.at[isb.at[b, 0]], sem_s, add=True)
        return carry

    lax.fori_loop(0, NCHUNK, chunk, 0)
    wait_scatter()
    plsc.subcore_barrier()

    pltpu.sync_copy(acc.at[pl.ds(s * RPT, RPT)], agg.at[c, pl.ds(s * RPT, RPT)])

    @pl.when(s == NS - 1)
    def _():
        pltpu.sync_copy(acc.at[pl.ds(TAIL_BASE, TAIL)],
                        agg.at[c, pl.ds(TAIL_BASE, TAIL)])

    pltpu.sync_copy(par.at[wid], pidx)
    for i in range(PAR_CHUNKS):
        pltpu.async_copy(items.at[pidx.at[i]], rows.at[0], sem_g).wait()
        pltpu.sync_copy(rows.at[0],
                        prow.at[pl.ds(wid * PAR_PER_W + i * CHUNK, CHUNK)])


def _sc_aggregate(items, ops, srcs, dsts, par, zeros):
    mesh = plsc.VectorSubcoreMesh(core_axis_name="c", subcore_axis_name="s")
    f = pl.kernel(
        _sc_body,
        out_type=(
            jax.ShapeDtypeStruct((NC, N, D), jnp.float32),
            jax.ShapeDtypeStruct((NPAR_PAD, D), jnp.float32),
        ),
        mesh=mesh,
        scratch_types=[
            pltpu.VMEM((2, 1, CHUNK), jnp.int32),
            pltpu.VMEM((2, 1, CHUNK), jnp.int32),
            pltpu.VMEM((2, CHUNK, D), jnp.float32),
            pltpu.VMEM((PAR_CHUNKS, CHUNK), jnp.int32),
            pltpu.VMEM_SHARED((N, D), jnp.float32),
            pltpu.SemaphoreType.DMA,
            pltpu.SemaphoreType.DMA,
            pltpu.SemaphoreType.DMA,
        ],
    )
    return f(items, ops, srcs, dsts, par, zeros)


def _pad_edges(edge_row, fill):
    x = edge_row.reshape(NS, EPT)
    x = jnp.pad(x, ((0, 0), (0, EPT_PAD - EPT)), constant_values=fill)
    return x.reshape(NS * NCHUNK, 1, CHUNK).astype(jnp.int32)


BLK = 2000


def _mlp_body(items, prow, aggc, aggo,
              ws1, bs1, ws2, bs2, wp1, bp1, wp2, bp2,
              wch1, bch1, wch2, bch2, wo1, bo1, wo2, bo2,
              wc1, bc1, wc2, bc2, wc3, bc3, out):
    prec = lax.Precision.HIGHEST

    def mm(x, w):
        return lax.dot_general(x, w, (((1,), (0,)), ((), ())),
                               precision=prec,
                               preferred_element_type=jnp.float32)

    def mlp2(x, w1, b1, w2, b2):
        return mm(jnp.maximum(mm(x, w1) + b1, 0.0), w2) + b2

    se = mlp2(items[...], ws1[...], bs1[...], ws2[...], bs2[...])
    pe = mlp2(prow[...], wp1[...], bp1[...], wp2[...], bp2[...])
    ce = mlp2(aggc[...], wch1[...], bch1[...], wch2[...], bch2[...])
    oe = mlp2(aggo[...], wo1[...], bo1[...], wo2[...], bo2[...])

    w = wc1[...]
    h = jnp.maximum(mm(pe, w[0:D]) + mm(ce, w[D:2 * D])
                    + mm(oe, w[2 * D:3 * D]) + mm(se, w[3 * D:4 * D])
                    + bc1[...], 0.0)
    h = jnp.maximum(mm(h, wc2[...]) + bc2[...], 0.0)
    o = mm(h, wc3[...]) + bc3[...]

    row = lax.broadcasted_iota(jnp.int32, (BLK, 1), 0) + pl.program_id(0) * BLK
    out[...] = jnp.where(row == N - 1, 0.0, o)


def _dense(items, prow, aggc, aggo, W):
    rowspec = pl.BlockSpec((BLK, D), lambda i: (i, 0))

    def fullspec(shape):
        return pl.BlockSpec(shape, lambda i: tuple(0 for _ in shape))

    wspecs = []
    wvals = []
    for w in W:
        if w.ndim == 1:
            w = w.reshape(1, -1)
        wvals.append(w)
        wspecs.append(fullspec(w.shape))

    return pl.pallas_call(
        _mlp_body,
        grid=(N // BLK,),
        in_specs=[rowspec, rowspec, rowspec, rowspec] + wspecs,
        out_specs=rowspec,
        out_shape=jax.ShapeDtypeStruct((N, D), jnp.float32),
    )(items, prow, aggc, aggo, *wvals)


def kernel(items, parents, operations, item_edge_index, op_edge_index,
           Ws1, bs1, Ws2, bs2, Wp1, bp1, Wp2, bp2, Wch1, bch1, Wch2, bch2,
           Wo1, bo1, Wo2, bo2, Wc1, bc1, Wc2, bc2, Wc3, bc3):
    srcs = jnp.concatenate([_pad_edges(item_edge_index[0], N - 1),
                            _pad_edges(op_edge_index[0], N - 1)])
    dsts = jnp.concatenate([_pad_edges(item_edge_index[1], 0),
                            _pad_edges(op_edge_index[1], 0)])
    par = jnp.pad(parents.astype(jnp.int32), (0, NPAR_PAD - N))
    par = par.reshape(NC * NS, PAR_CHUNKS, CHUNK)
    zeros = jnp.zeros((RPT, D), jnp.float32)

    agg, prow = _sc_aggregate(items, operations, srcs, dsts, par, zeros)
    aggc, aggo = agg[0], agg[1]
    prow = prow[:N]

    W = (Ws1, bs1, Ws2, bs2, Wp1, bp1, Wp2, bp2, Wch1, bch1, Wch2, bch2,
         Wo1, bo1, Wo2, bo2, Wc1, bc1, Wc2, bc2, Wc3, bc3)
    return _dense(items, prow, aggc, aggo, W)


# speedup vs baseline: 4.9362x; 1.1113x over previous
import jax
import jax.numpy as jnp
from jax import lax
from jax.experimental import pallas as pl
from jax.experimental.pallas import tpu as pltpu
from jax.experimental.pallas import tpu_sc as plsc

N = 10000
D = 128
E = 320000
NC = 2
NS = 16
CHUNK = 128
EPT = E // NS
NCHUNK = 158
EPT_PAD = NCHUNK * CHUNK
RPT = 624
TAIL_BASE = NS * RPT
TAIL = N - TAIL_BASE
PAR_CHUNKS = 3
PAR_PER_W = PAR_CHUNKS * CHUNK
NPAR_PAD = NC * NS * PAR_PER_W
NIDX = 4
NROW = 3


def _sc_body(items, ops, srcs, dsts, par, zeros, agg, prow,
             isb, idb, rows, pidx, acc, sem_i, sem_g, sem_s):
    c = lax.axis_index("c")
    s = lax.axis_index("s")
    wid = c * NS + s
    base_r = wid * NCHUNK

    pltpu.sync_copy(zeros, acc.at[pl.ds(s * RPT, RPT)])

    @pl.when(s == NS - 1)
    def _():
        pltpu.sync_copy(zeros.at[pl.ds(0, TAIL)], acc.at[pl.ds(TAIL_BASE, TAIL)])

    plsc.subcore_barrier()

    def issue_idx(j):
        b = j % NIDX
        pltpu.async_copy(srcs.at[base_r + j], isb.at[b], sem_i)
        pltpu.async_copy(dsts.at[base_r + j], idb.at[b], sem_i)

    def wait_idx():
        pltpu.make_async_copy(srcs.at[0], isb.at[0], sem_i).wait()
        pltpu.make_async_copy(dsts.at[0], idb.at[0], sem_i).wait()

    def issue_gather(j):
        b = j % NROW

        @pl.when(c == 0)
        def _():
            pltpu.async_copy(items.at[idb.at[j % NIDX, 0]], rows.at[b], sem_g)

        @pl.when(c == 1)
        def _():
            pltpu.async_copy(ops.at[idb.at[j % NIDX, 0]], rows.at[b], sem_g)

    def wait_gather():
        pltpu.make_async_copy(items.at[idb.at[0, 0]], rows.at[0], sem_g).wait()

    def wait_scatter():
        pltpu.make_async_copy(rows.at[0], acc.at[isb.at[0, 0]], sem_s).wait()

    issue_idx(0)
    issue_idx(1)
    wait_idx()
    issue_gather(0)

    def chunk(j, carry):
        @pl.when(j >= 2)
        def _():
            wait_scatter()

        @pl.when(j + 2 < NCHUNK)
        def _():
            issue_idx(j + 2)

        @pl.when(j + 1 < NCHUNK)
        def _():
            wait_idx()
            issue_gather(j + 1)

        wait_gather()
        pltpu.async_copy(rows.at[j % NROW], acc.at[isb.at[j % NIDX, 0]],
                         sem_s, add=True)
        return carry

    lax.fori_loop(0, NCHUNK, chunk, 0)
    wait_scatter()
    wait_scatter()
    plsc.subcore_barrier()

    pltpu.sync_copy(acc.at[pl.ds(s * RPT, RPT)], agg.at[c, pl.ds(s * RPT, RPT)])

    @pl.when(s == NS - 1)
    def _():
        pltpu.sync_copy(acc.at[pl.ds(TAIL_BASE, TAIL)],
                        agg.at[c, pl.ds(TAIL_BASE, TAIL)])

    pltpu.sync_copy(par.at[wid], pidx)
    for i in range(PAR_CHUNKS):
        pltpu.async_copy(items.at[pidx.at[i]], rows.at[0], sem_g).wait()
        pltpu.sync_copy(rows.at[0],
                        prow.at[pl.ds(wid * PAR_PER_W + i * CHUNK, CHUNK)])


def _sc_aggregate(items, ops, srcs, dsts, par, zeros):
    mesh = plsc.VectorSubcoreMesh(core_axis_name="c", subcore_axis_name="s")
    f = pl.kernel(
        _sc_body,
        out_type=(
            jax.ShapeDtypeStruct((NC, N, D), jnp.float32),
            jax.ShapeDtypeStruct((NPAR_PAD, D), jnp.float32),
        ),
        mesh=mesh,
        scratch_types=[
            pltpu.VMEM((NIDX, 1, CHUNK), jnp.int32),
            pltpu.VMEM((NIDX, 1, CHUNK), jnp.int32),
            pltpu.VMEM((NROW, CHUNK, D), jnp.float32),
            pltpu.VMEM((PAR_CHUNKS, CHUNK), jnp.int32),
            pltpu.VMEM_SHARED((N, D), jnp.float32),
            pltpu.SemaphoreType.DMA,
            pltpu.SemaphoreType.DMA,
            pltpu.SemaphoreType.DMA,
        ],
    )
    return f(items, ops, srcs, dsts, par, zeros)


def _pad_edges(edge_row, fill):
    x = edge_row.reshape(NS, EPT)
    x = jnp.pad(x, ((0, 0), (0, EPT_PAD - EPT)), constant_values=fill)
    return x.reshape(NS * NCHUNK, 1, CHUNK).astype(jnp.int32)


BLK = 2000


def _mlp_body(items, prow, aggc, aggo,
              ws1, bs1, ws2, bs2, wp1, bp1, wp2, bp2,
              wch1, bch1, wch2, bch2, wo1, bo1, wo2, bo2,
              wc1, bc1, wc2, bc2, wc3, bc3, out):
    prec = lax.Precision.HIGHEST

    def mm(x, w):
        return lax.dot_general(x, w, (((1,), (0,)), ((), ())),
                               precision=prec,
                               preferred_element_type=jnp.float32)

    def mlp2(x, w1, b1, w2, b2):
        return mm(jnp.maximum(mm(x, w1) + b1, 0.0), w2) + b2

    se = mlp2(items[...], ws1[...], bs1[...], ws2[...], bs2[...])
    pe = mlp2(prow[...], wp1[...], bp1[...], wp2[...], bp2[...])
    ce = mlp2(aggc[...], wch1[...], bch1[...], wch2[...], bch2[...])
    oe = mlp2(aggo[...], wo1[...], bo1[...], wo2[...], bo2[...])

    w = wc1[...]
    h = jnp.maximum(mm(pe, w[0:D]) + mm(ce, w[D:2 * D])
                    + mm(oe, w[2 * D:3 * D]) + mm(se, w[3 * D:4 * D])
                    + bc1[...], 0.0)
    h = jnp.maximum(mm(h, wc2[...]) + bc2[...], 0.0)
    o = mm(h, wc3[...]) + bc3[...]

    row = lax.broadcasted_iota(jnp.int32, (BLK, 1), 0) + pl.program_id(0) * BLK
    out[...] = jnp.where(row == N - 1, 0.0, o)


def _dense(items, prow, aggc, aggo, W):
    rowspec = pl.BlockSpec((BLK, D), lambda i: (i, 0))

    def fullspec(shape):
        return pl.BlockSpec(shape, lambda i: tuple(0 for _ in shape))

    wspecs = []
    wvals = []
    for w in W:
        if w.ndim == 1:
            w = w.reshape(1, -1)
        wvals.append(w)
        wspecs.append(fullspec(w.shape))

    return pl.pallas_call(
        _mlp_body,
        grid=(N // BLK,),
        in_specs=[rowspec, rowspec, rowspec, rowspec] + wspecs,
        out_specs=rowspec,
        out_shape=jax.ShapeDtypeStruct((N, D), jnp.float32),
    )(items, prow, aggc, aggo, *wvals)


def kernel(items, parents, operations, item_edge_index, op_edge_index,
           Ws1, bs1, Ws2, bs2, Wp1, bp1, Wp2, bp2, Wch1, bch1, Wch2, bch2,
           Wo1, bo1, Wo2, bo2, Wc1, bc1, Wc2, bc2, Wc3, bc3):
    srcs = jnp.concatenate([_pad_edges(item_edge_index[0], N - 1),
                            _pad_edges(op_edge_index[0], N - 1)])
    dsts = jnp.concatenate([_pad_edges(item_edge_index[1], 0),
                            _pad_edges(op_edge_index[1], 0)])
    par = jnp.pad(parents.astype(jnp.int32), (0, NPAR_PAD - N))
    par = par.reshape(NC * NS, PAR_CHUNKS, CHUNK)
    zeros = jnp.zeros((RPT, D), jnp.float32)

    agg, prow = _sc_aggregate(items, operations, srcs, dsts, par, zeros)
    aggc, aggo = agg[0], agg[1]
    prow = prow[:N]

    W = (Ws1, bs1, Ws2, bs2, Wp1, bp1, Wp2, bp2, Wch1, bch1, Wch2, bch2,
         Wo1, bo1, Wo2, bo2, Wc1, bc1, Wc2, bc2, Wc3, bc3)
    return _dense(items, prow, aggc, aggo, W)


# speedup vs baseline: 8.7426x; 1.7711x over previous
import jax
import jax.numpy as jnp
from jax import lax
from jax.experimental import pallas as pl
from jax.experimental.pallas import tpu as pltpu
from jax.experimental.pallas import tpu_sc as plsc

N = 10000
D = 128
E = 320000
NC = 2
NS = 16
CHUNK = 128
ROWS_E = E // CHUNK
CH_LO = ROWS_E // NS
CH_EXTRA = ROWS_E - CH_LO * NS
RPT = 624
TAIL_BASE = NS * RPT
TAIL = N - TAIL_BASE
PAR_CHUNKS = 3
PAR_PER_W = PAR_CHUNKS * CHUNK
NPAR_PAD = NC * NS * PAR_PER_W
NIDX = 4
NROW = 3


def _sc_body(items, ops, isrc, idst, osrc, odst, par, zeros, agg, prow,
             isb, idb, rows, pidx, acc, sem_i, sem_g, sem_s):
    c = lax.axis_index("c")
    s = lax.axis_index("s")
    wid = c * NS + s
    trips = jnp.where(s < CH_EXTRA, CH_LO + 1, CH_LO)

    pltpu.sync_copy(zeros, acc.at[pl.ds(s * RPT, RPT)])

    @pl.when(s == NS - 1)
    def _():
        pltpu.sync_copy(zeros.at[pl.ds(0, TAIL)], acc.at[pl.ds(TAIL_BASE, TAIL)])

    plsc.subcore_barrier()

    def issue_idx(j):
        b = j % NIDX
        row = s + NS * j

        @pl.when(c == 0)
        def _():
            pltpu.async_copy(isrc.at[row], isb.at[b], sem_i)
            pltpu.async_copy(idst.at[row], idb.at[b], sem_i)

        @pl.when(c == 1)
        def _():
            pltpu.async_copy(osrc.at[row], isb.at[b], sem_i)
            pltpu.async_copy(odst.at[row], idb.at[b], sem_i)

    def wait_idx():
        pltpu.make_async_copy(isrc.at[0], isb.at[0], sem_i).wait()
        pltpu.make_async_copy(isrc.at[0], idb.at[0], sem_i).wait()

    def issue_gather(j):
        b = j % NROW

        @pl.when(c == 0)
        def _():
            pltpu.async_copy(items.at[idb.at[j % NIDX, 0]], rows.at[b], sem_g)

        @pl.when(c == 1)
        def _():
            pltpu.async_copy(ops.at[idb.at[j % NIDX, 0]], rows.at[b], sem_g)

    def wait_gather():
        pltpu.make_async_copy(items.at[idb.at[0, 0]], rows.at[0], sem_g).wait()

    def wait_scatter():
        pltpu.make_async_copy(rows.at[0], acc.at[isb.at[0, 0]], sem_s).wait()

    issue_idx(0)
    issue_idx(1)
    wait_idx()
    issue_gather(0)

    def chunk(j, carry):
        @pl.when(j >= 2)
        def _():
            wait_scatter()

        @pl.when(j + 2 < trips)
        def _():
            issue_idx(j + 2)

        @pl.when(j + 1 < trips)
        def _():
            wait_idx()
            issue_gather(j + 1)

        wait_gather()
        pltpu.async_copy(rows.at[j % NROW], acc.at[isb.at[j % NIDX, 0]],
                         sem_s, add=True)
        return carry

    lax.fori_loop(0, trips, chunk, 0)
    wait_scatter()
    wait_scatter()
    plsc.subcore_barrier()

    pltpu.sync_copy(acc.at[pl.ds(s * RPT, RPT)], agg.at[c, pl.ds(s * RPT, RPT)])

    @pl.when(s == NS - 1)
    def _():
        pltpu.sync_copy(acc.at[pl.ds(TAIL_BASE, TAIL)],
                        agg.at[c, pl.ds(TAIL_BASE, TAIL)])

    pltpu.sync_copy(par.at[wid], pidx)
    for i in range(PAR_CHUNKS):
        pltpu.async_copy(items.at[pidx.at[i]], rows.at[0], sem_g).wait()
        pltpu.sync_copy(rows.at[0],
                        prow.at[pl.ds(wid * PAR_PER_W + i * CHUNK, CHUNK)])


def _sc_aggregate(items, ops, isrc, idst, osrc, odst, par, zeros):
    mesh = plsc.VectorSubcoreMesh(core_axis_name="c", subcore_axis_name="s")
    f = pl.kernel(
        _sc_body,
        out_type=(
            jax.ShapeDtypeStruct((NC, N, D), jnp.float32),
            jax.ShapeDtypeStruct((NPAR_PAD, D), jnp.float32),
        ),
        mesh=mesh,
        scratch_types=[
            pltpu.VMEM((NIDX, 1, CHUNK), jnp.int32),
            pltpu.VMEM((NIDX, 1, CHUNK), jnp.int32),
            pltpu.VMEM((NROW, CHUNK, D), jnp.float32),
            pltpu.VMEM((PAR_CHUNKS, CHUNK), jnp.int32),
            pltpu.VMEM_SHARED((N, D), jnp.float32),
            pltpu.SemaphoreType.DMA,
            pltpu.SemaphoreType.DMA,
            pltpu.SemaphoreType.DMA,
        ],
    )
    return f(items, ops, isrc, idst, osrc, odst, par, zeros)


BLK = 2000


def _mlp_body(items, prow, aggc, aggo,
              ws1, bs1, ws2, bs2, wp1, bp1, wp2, bp2,
              wch1, bch1, wch2, bch2, wo1, bo1, wo2, bo2,
              wc1, bc1, wc2, bc2, wc3, bc3, out):
    prec = lax.Precision.DEFAULT

    def mm(x, w):
        return lax.dot_general(x, w, (((1,), (0,)), ((), ())),
                               precision=prec,
                               preferred_element_type=jnp.float32)

    def mlp2(x, w1, b1, w2, b2):
        return mm(jnp.maximum(mm(x, w1) + b1, 0.0), w2) + b2

    se = mlp2(items[...], ws1[...], bs1[...], ws2[...], bs2[...])
    pe = mlp2(prow[...], wp1[...], bp1[...], wp2[...], bp2[...])
    ce = mlp2(aggc[...], wch1[...], bch1[...], wch2[...], bch2[...])
    oe = mlp2(aggo[...], wo1[...], bo1[...], wo2[...], bo2[...])

    w = wc1[...]
    h = jnp.maximum(mm(pe, w[0:D]) + mm(ce, w[D:2 * D])
                    + mm(oe, w[2 * D:3 * D]) + mm(se, w[3 * D:4 * D])
                    + bc1[...], 0.0)
    h = jnp.maximum(mm(h, wc2[...]) + bc2[...], 0.0)
    o = mm(h, wc3[...]) + bc3[...]

    row = lax.broadcasted_iota(jnp.int32, (BLK, 1), 0) + pl.program_id(0) * BLK
    out[...] = jnp.where(row == N - 1, 0.0, o)


def _dense(items, prow, aggc, aggo, W):
    rowspec = pl.BlockSpec((BLK, D), lambda i: (i, 0))

    def fullspec(shape):
        return pl.BlockSpec(shape, lambda i: tuple(0 for _ in shape))

    wspecs = []
    wvals = []
    for w in W:
        if w.ndim == 1:
            w = w.reshape(1, -1)
        wvals.append(w)
        wspecs.append(fullspec(w.shape))

    return pl.pallas_call(
        _mlp_body,
        grid=(N // BLK,),
        in_specs=[rowspec, rowspec, rowspec, rowspec] + wspecs,
        out_specs=rowspec,
        out_shape=jax.ShapeDtypeStruct((N, D), jnp.float32),
    )(items, prow, aggc, aggo, *wvals)


def kernel(items, parents, operations, item_edge_index, op_edge_index,
           Ws1, bs1, Ws2, bs2, Wp1, bp1, Wp2, bp2, Wch1, bch1, Wch2, bch2,
           Wo1, bo1, Wo2, bo2, Wc1, bc1, Wc2, bc2, Wc3, bc3):
    isrc = item_edge_index[0].astype(jnp.int32).reshape(ROWS_E, 1, CHUNK)
    idst = item_edge_index[1].astype(jnp.int32).reshape(ROWS_E, 1, CHUNK)
    osrc = op_edge_index[0].astype(jnp.int32).reshape(ROWS_E, 1, CHUNK)
    odst = op_edge_index[1].astype(jnp.int32).reshape(ROWS_E, 1, CHUNK)
    par = jnp.pad(parents.astype(jnp.int32), (0, NPAR_PAD - N))
    par = par.reshape(NC * NS, PAR_CHUNKS, CHUNK)
    zeros = jnp.zeros((RPT, D), jnp.float32)

    agg, prow = _sc_aggregate(items, operations, isrc, idst, osrc, odst,
                              par, zeros)
    aggc, aggo = agg[0], agg[1]
    prow = prow[:N]

    W = (Ws1, bs1, Ws2, bs2, Wp1, bp1, Wp2, bp2, Wch1, bch1, Wch2, bch2,
         Wo1, bo1, Wo2, bo2, Wc1, bc1, Wc2, bc2, Wc3, bc3)
    return _dense(items, prow, aggc, aggo, W)


# speedup vs baseline: 8.7463x; 1.0004x over previous
import jax
import jax.numpy as jnp
from jax import lax
from jax.experimental import pallas as pl
from jax.experimental.pallas import tpu as pltpu
from jax.experimental.pallas import tpu_sc as plsc

N = 10000
D = 128
E = 320000
NC = 2
NS = 16
CHUNK = 128
ROWS_E = E // CHUNK
CH_LO = ROWS_E // NS
CH_EXTRA = ROWS_E - CH_LO * NS
RPT = 624
TAIL_BASE = NS * RPT
TAIL = N - TAIL_BASE
PAR_CHUNKS = 3
PAR_PER_W = PAR_CHUNKS * CHUNK
NPAR_PAD = NC * NS * PAR_PER_W
NIDX = 4
NROW = 3


def _sc_body(items, ops, isrc, idst, osrc, odst, par, zeros, agg, prow,
             isb, idb, rows, pidx, acc, sem_i, sem_g, sem_s):
    c = lax.axis_index("c")
    s = lax.axis_index("s")
    wid = c * NS + s
    trips = jnp.where(s < CH_EXTRA, CH_LO + 1, CH_LO)

    pltpu.sync_copy(zeros, acc.at[pl.ds(s * RPT, RPT)])

    @pl.when(s == NS - 1)
    def _():
        pltpu.sync_copy(zeros.at[pl.ds(0, TAIL)], acc.at[pl.ds(TAIL_BASE, TAIL)])

    plsc.subcore_barrier()

    def issue_idx(j):
        b = j % NIDX
        row = s + NS * j

        @pl.when(c == 1)
        def _():
            pltpu.async_copy(isrc.at[row], isb.at[b], sem_i)
            pltpu.async_copy(idst.at[row], idb.at[b], sem_i)

        @pl.when(c == 0)
        def _():
            pltpu.async_copy(osrc.at[row], isb.at[b], sem_i)
            pltpu.async_copy(odst.at[row], idb.at[b], sem_i)

    def wait_idx():
        pltpu.make_async_copy(isrc.at[0], isb.at[0], sem_i).wait()
        pltpu.make_async_copy(isrc.at[0], idb.at[0], sem_i).wait()

    def issue_gather(j):
        b = j % NROW

        @pl.when(c == 1)
        def _():
            pltpu.async_copy(items.at[idb.at[j % NIDX, 0]], rows.at[b], sem_g)

        @pl.when(c == 0)
        def _():
            pltpu.async_copy(ops.at[idb.at[j % NIDX, 0]], rows.at[b], sem_g)

    def wait_gather():
        pltpu.make_async_copy(items.at[idb.at[0, 0]], rows.at[0], sem_g).wait()

    def wait_scatter():
        pltpu.make_async_copy(rows.at[0], acc.at[isb.at[0, 0]], sem_s).wait()

    issue_idx(0)
    issue_idx(1)
    wait_idx()
    issue_gather(0)

    def chunk(j, carry):
        @pl.when(j >= 2)
        def _():
            wait_scatter()

        @pl.when(j + 2 < trips)
        def _():
            issue_idx(j + 2)

        @pl.when(j + 1 < trips)
        def _():
            wait_idx()
            issue_gather(j + 1)

        wait_gather()
        pltpu.async_copy(rows.at[j % NROW], acc.at[isb.at[j % NIDX, 0]],
                         sem_s, add=True)
        return carry

    lax.fori_loop(0, trips, chunk, 0)
    wait_scatter()
    wait_scatter()
    plsc.subcore_barrier()

    pltpu.sync_copy(acc.at[pl.ds(s * RPT, RPT)], agg.at[1 - c, pl.ds(s * RPT, RPT)])

    @pl.when(s == NS - 1)
    def _():
        pltpu.sync_copy(acc.at[pl.ds(TAIL_BASE, TAIL)],
                        agg.at[1 - c, pl.ds(TAIL_BASE, TAIL)])

    pltpu.sync_copy(par.at[wid], pidx)
    for i in range(PAR_CHUNKS):
        pltpu.async_copy(items.at[pidx.at[i]], rows.at[0], sem_g).wait()
        pltpu.sync_copy(rows.at[0],
                        prow.at[pl.ds(wid * PAR_PER_W + i * CHUNK, CHUNK)])


def _sc_aggregate(items, ops, isrc, idst, osrc, odst, par, zeros):
    mesh = plsc.VectorSubcoreMesh(core_axis_name="c", subcore_axis_name="s")
    f = pl.kernel(
        _sc_body,
        out_type=(
            jax.ShapeDtypeStruct((NC, N, D), jnp.float32),
            jax.ShapeDtypeStruct((NPAR_PAD, D), jnp.float32),
        ),
        mesh=mesh,
        scratch_types=[
            pltpu.VMEM((NIDX, 1, CHUNK), jnp.int32),
            pltpu.VMEM((NIDX, 1, CHUNK), jnp.int32),
            pltpu.VMEM((NROW, CHUNK, D), jnp.float32),
            pltpu.VMEM((PAR_CHUNKS, CHUNK), jnp.int32),
            pltpu.VMEM_SHARED((N, D), jnp.float32),
            pltpu.SemaphoreType.DMA,
            pltpu.SemaphoreType.DMA,
            pltpu.SemaphoreType.DMA,
        ],
    )
    return f(items, ops, isrc, idst, osrc, odst, par, zeros)


BLK = 2000


def _mlp_body(items, prow, aggc, aggo,
              ws1, bs1, ws2, bs2, wp1, bp1, wp2, bp2,
              wch1, bch1, wch2, bch2, wo1, bo1, wo2, bo2,
              wc1, bc1, wc2, bc2, wc3, bc3, out):
    prec = lax.Precision.DEFAULT

    def mm(x, w):
        return lax.dot_general(x, w, (((1,), (0,)), ((), ())),
                               precision=prec,
                               preferred_element_type=jnp.float32)

    def mlp2(x, w1, b1, w2, b2):
        return mm(jnp.maximum(mm(x, w1) + b1, 0.0), w2) + b2

    se = mlp2(items[...], ws1[...], bs1[...], ws2[...], bs2[...])
    pe = mlp2(prow[...], wp1[...], bp1[...], wp2[...], bp2[...])
    ce = mlp2(aggc[...], wch1[...], bch1[...], wch2[...], bch2[...])
    oe = mlp2(aggo[...], wo1[...], bo1[...], wo2[...], bo2[...])

    w = wc1[...]
    h = jnp.maximum(mm(pe, w[0:D]) + mm(ce, w[D:2 * D])
                    + mm(oe, w[2 * D:3 * D]) + mm(se, w[3 * D:4 * D])
                    + bc1[...], 0.0)
    h = jnp.maximum(mm(h, wc2[...]) + bc2[...], 0.0)
    o = mm(h, wc3[...]) + bc3[...]

    row = lax.broadcasted_iota(jnp.int32, (BLK, 1), 0) + pl.program_id(0) * BLK
    out[...] = jnp.where(row == N - 1, 0.0, o)


def _dense(items, prow, aggc, aggo, W):
    rowspec = pl.BlockSpec((BLK, D), lambda i: (i, 0))

    def fullspec(shape):
        return pl.BlockSpec(shape, lambda i: tuple(0 for _ in shape))

    wspecs = []
    wvals = []
    for w in W:
        if w.ndim == 1:
            w = w.reshape(1, -1)
        wvals.append(w)
        wspecs.append(fullspec(w.shape))

    return pl.pallas_call(
        _mlp_body,
        grid=(N // BLK,),
        in_specs=[rowspec, rowspec, rowspec, rowspec] + wspecs,
        out_specs=rowspec,
        out_shape=jax.ShapeDtypeStruct((N, D), jnp.float32),
    )(items, prow, aggc, aggo, *wvals)


def kernel(items, parents, operations, item_edge_index, op_edge_index,
           Ws1, bs1, Ws2, bs2, Wp1, bp1, Wp2, bp2, Wch1, bch1, Wch2, bch2,
           Wo1, bo1, Wo2, bo2, Wc1, bc1, Wc2, bc2, Wc3, bc3):
    isrc = item_edge_index[0].astype(jnp.int32).reshape(ROWS_E, 1, CHUNK)
    idst = item_edge_index[1].astype(jnp.int32).reshape(ROWS_E, 1, CHUNK)
    osrc = op_edge_index[0].astype(jnp.int32).reshape(ROWS_E, 1, CHUNK)
    odst = op_edge_index[1].astype(jnp.int32).reshape(ROWS_E, 1, CHUNK)
    par = jnp.pad(parents.astype(jnp.int32), (0, NPAR_PAD - N))
    par = par.reshape(NC * NS, PAR_CHUNKS, CHUNK)
    zeros = jnp.zeros((RPT, D), jnp.float32)

    agg, prow = _sc_aggregate(items, operations, isrc, idst, osrc, odst,
                              par, zeros)
    aggc, aggo = agg[0], agg[1]
    prow = prow[:N]

    W = (Ws1, bs1, Ws2, bs2, Wp1, bp1, Wp2, bp2, Wch1, bch1, Wch2, bch2,
         Wo1, bo1, Wo2, bo2, Wc1, bc1, Wc2, bc2, Wc3, bc3)
    return _dense(items, prow, aggc, aggo, W)


# speedup vs baseline: 12.4733x; 1.4261x over previous
import jax
import jax.numpy as jnp
from jax import lax
from jax.experimental import pallas as pl
from jax.experimental.pallas import tpu as pltpu
from jax.experimental.pallas import tpu_sc as plsc

N = 10000
D = 128
E = 320000
NC = 2
NS = 16
CHUNK = 128
ROWS_E = E // CHUNK
CH_LO = ROWS_E // NS
CH_EXTRA = ROWS_E - CH_LO * NS
RPT = 624
TAIL_BASE = NS * RPT
TAIL = N - TAIL_BASE
NPAR_FULL = N // CHUNK
NIDX = 4
NROW = 3


def _sc_body(items, ops, iedge, oedge, par, zeros, agg, prow,
             isb, idb, rows, pidx, acc, sem_i, sem_g, sem_s):
    c = lax.axis_index("c")
    s = lax.axis_index("s")
    wid = c * NS + s
    trips = jnp.where(s < CH_EXTRA, CH_LO + 1, CH_LO)

    pltpu.sync_copy(zeros, acc.at[pl.ds(s * RPT, RPT)])

    @pl.when(s == NS - 1)
    def _():
        pltpu.sync_copy(zeros.at[pl.ds(0, TAIL)], acc.at[pl.ds(TAIL_BASE, TAIL)])

    plsc.subcore_barrier()

    def issue_idx(j):
        b = j % NIDX
        row = s + NS * j

        @pl.when(c == 0)
        def _():
            pltpu.async_copy(iedge.at[0, row], isb.at[b], sem_i)
            pltpu.async_copy(iedge.at[1, row], idb.at[b], sem_i)

        @pl.when(c == 1)
        def _():
            pltpu.async_copy(oedge.at[0, row], isb.at[b], sem_i)
            pltpu.async_copy(oedge.at[1, row], idb.at[b], sem_i)

    def wait_idx():
        pltpu.make_async_copy(iedge.at[0, 0], isb.at[0], sem_i).wait()
        pltpu.make_async_copy(iedge.at[0, 0], idb.at[0], sem_i).wait()

    def issue_gather(j):
        b = j % NROW

        @pl.when(c == 0)
        def _():
            pltpu.async_copy(items.at[idb.at[j % NIDX, 0]], rows.at[b], sem_g)

        @pl.when(c == 1)
        def _():
            pltpu.async_copy(ops.at[idb.at[j % NIDX, 0]], rows.at[b], sem_g)

    def wait_gather():
        pltpu.make_async_copy(items.at[idb.at[0, 0]], rows.at[0], sem_g).wait()

    def wait_scatter():
        pltpu.make_async_copy(rows.at[0], acc.at[isb.at[0, 0]], sem_s).wait()

    issue_idx(0)
    issue_idx(1)
    wait_idx()
    issue_gather(0)

    def chunk(j, carry):
        @pl.when(j >= 2)
        def _():
            wait_scatter()

        @pl.when(j + 2 < trips)
        def _():
            issue_idx(j + 2)

        @pl.when(j + 1 < trips)
        def _():
            wait_idx()
            issue_gather(j + 1)

        wait_gather()
        pltpu.async_copy(rows.at[j % NROW], acc.at[isb.at[j % NIDX, 0]],
                         sem_s, add=True)
        return carry

    lax.fori_loop(0, trips, chunk, 0)
    wait_scatter()
    wait_scatter()
    plsc.subcore_barrier()

    pltpu.sync_copy(acc.at[pl.ds(s * RPT, RPT)], agg.at[c, pl.ds(s * RPT, RPT)])

    @pl.when(s == NS - 1)
    def _():
        pltpu.sync_copy(acc.at[pl.ds(TAIL_BASE, TAIL)],
                        agg.at[c, pl.ds(TAIL_BASE, TAIL)])

    def par_chunk(p):
        pltpu.sync_copy(par.at[pl.ds(p * CHUNK, CHUNK)], pidx)
        pltpu.async_copy(items.at[pidx], rows.at[0], sem_g).wait()
        pltpu.sync_copy(rows.at[0], prow.at[pl.ds(p * CHUNK, CHUNK)])

    par_chunk(wid)
    par_chunk(wid + NC * NS)

    @pl.when(wid < NPAR_FULL - 2 * NC * NS)
    def _():
        par_chunk(wid + 2 * NC * NS)

    @pl.when(wid == NC * NS - 1)
    def _():
        pltpu.sync_copy(par.at[pl.ds(TAIL_BASE, TAIL)], pidx.at[pl.ds(0, TAIL)])
        pltpu.async_copy(items.at[pidx.at[pl.ds(0, TAIL)]],
                         rows.at[0, pl.ds(0, TAIL)], sem_g).wait()
        pltpu.sync_copy(rows.at[0, pl.ds(0, TAIL)],
                        prow.at[pl.ds(TAIL_BASE, TAIL)])


def _sc_aggregate(items, ops, iedge, oedge, par, zeros):
    mesh = plsc.VectorSubcoreMesh(core_axis_name="c", subcore_axis_name="s")
    f = pl.kernel(
        _sc_body,
        out_type=(
            jax.ShapeDtypeStruct((NC, N, D), jnp.float32),
            jax.ShapeDtypeStruct((N, D), jnp.float32),
        ),
        mesh=mesh,
        scratch_types=[
            pltpu.VMEM((NIDX, 1, CHUNK), jnp.int32),
            pltpu.VMEM((NIDX, 1, CHUNK), jnp.int32),
            pltpu.VMEM((NROW, CHUNK, D), jnp.float32),
            pltpu.VMEM((CHUNK,), jnp.int32),
            pltpu.VMEM_SHARED((N, D), jnp.float32),
            pltpu.SemaphoreType.DMA,
            pltpu.SemaphoreType.DMA,
            pltpu.SemaphoreType.DMA,
        ],
    )
    return f(items, ops, iedge, oedge, par, zeros)


BLK = 2000


def _mlp_body(items, prow, aggc, aggo,
              ws1, bs1, ws2, bs2, wp1, bp1, wp2, bp2,
              wch1, bch1, wch2, bch2, wo1, bo1, wo2, bo2,
              wc1, bc1, wc2, bc2, wc3, bc3, out):
    prec = lax.Precision.DEFAULT

    def mm(x, w):
        return lax.dot_general(x, w, (((1,), (0,)), ((), ())),
                               precision=prec,
                               preferred_element_type=jnp.float32)

    def mlp2(x, w1, b1, w2, b2):
        return mm(jnp.maximum(mm(x, w1) + b1, 0.0), w2) + b2

    se = mlp2(items[...], ws1[...], bs1[...], ws2[...], bs2[...])
    pe = mlp2(prow[...], wp1[...], bp1[...], wp2[...], bp2[...])
    ce = mlp2(aggc[0], wch1[...], bch1[...], wch2[...], bch2[...])
    oe = mlp2(aggo[0], wo1[...], bo1[...], wo2[...], bo2[...])

    w = wc1[...]
    h = jnp.maximum(mm(pe, w[0:D]) + mm(ce, w[D:2 * D])
                    + mm(oe, w[2 * D:3 * D]) + mm(se, w[3 * D:4 * D])
                    + bc1[...], 0.0)
    h = jnp.maximum(mm(h, wc2[...]) + bc2[...], 0.0)
    o = mm(h, wc3[...]) + bc3[...]

    row = lax.broadcasted_iota(jnp.int32, (BLK, 1), 0) + pl.program_id(0) * BLK
    out[...] = jnp.where(row == N - 1, 0.0, o)


def _dense(items, prow, agg, W):
    rowspec = pl.BlockSpec((BLK, D), lambda i: (i, 0))
    aggc_spec = pl.BlockSpec((1, BLK, D), lambda i: (0, i, 0))
    aggo_spec = pl.BlockSpec((1, BLK, D), lambda i: (1, i, 0))

    def fullspec(shape):
        return pl.BlockSpec(shape, lambda i: tuple(0 for _ in shape))

    wspecs = []
    wvals = []
    for w in W:
        if w.ndim == 1:
            w = w.reshape(1, -1)
        wvals.append(w)
        wspecs.append(fullspec(w.shape))

    return pl.pallas_call(
        _mlp_body,
        grid=(N // BLK,),
        in_specs=[rowspec, rowspec, aggc_spec, aggo_spec] + wspecs,
        out_specs=rowspec,
        out_shape=jax.ShapeDtypeStruct((N, D), jnp.float32),
    )(items, prow, agg, agg, *wvals)


def kernel(items, parents, operations, item_edge_index, op_edge_index,
           Ws1, bs1, Ws2, bs2, Wp1, bp1, Wp2, bp2, Wch1, bch1, Wch2, bch2,
           Wo1, bo1, Wo2, bo2, Wc1, bc1, Wc2, bc2, Wc3, bc3):
    iedge = item_edge_index.astype(jnp.int32).reshape(2, ROWS_E, 1, CHUNK)
    oedge = op_edge_index.astype(jnp.int32).reshape(2, ROWS_E, 1, CHUNK)
    par = parents.astype(jnp.int32)
    zeros = jnp.zeros((RPT, D), jnp.float32)

    agg, prow = _sc_aggregate(items, operations, iedge, oedge, par, zeros)

    W = (Ws1, bs1, Ws2, bs2, Wp1, bp1, Wp2, bp2, Wch1, bch1, Wch2, bch2,
         Wo1, bo1, Wo2, bo2, Wc1, bc1, Wc2, bc2, Wc3, bc3)
    return _dense(items, prow, agg, W)
